# initial kernel scaffold (unmeasured)
import jax
import jax.numpy as jnp
from jax import lax
from jax.experimental import pallas as pl
from jax.experimental.pallas import tpu as pltpu

T_CORR = 160


def kernel(x, A, B, C):
    Bb, S, D = x.shape
    N = A.shape[1]

    dAT = jnp.exp(A).T
    xT = x.transpose(1, 0, 2)
    Bq = B.transpose(1, 0, 2)[..., None]
    Cq = C.transpose(1, 0, 2)[..., None]

    def body(x_ref, da_ref, b_ref, c_ref, y_ref, h_ref, comm_ref,
             send_sem, recv_sem):
        my_x = lax.axis_index("x")
        my_y = lax.axis_index("y")

        h_ref[...] = jnp.zeros(h_ref.shape, h_ref.dtype)
        da = da_ref[...][None]

        def step(t, carry):
            h = h_ref[...] * da + x_ref[t][:, None, :] * b_ref[t]
            h_ref[...] = h
            y_ref[t] = (h * c_ref[t]).sum(axis=1)
            return carry

        lax.fori_loop(0, S, step, 0)

        rdma = pltpu.make_async_remote_copy(
            src_ref=h_ref,
            dst_ref=comm_ref,
            send_sem=send_sem,
            recv_sem=recv_sem,
            device_id=(1 - my_x, my_y),
            device_id_type=pl.DeviceIdType.MESH,
        )

        @pl.when(my_x == 0)
        def _():
            rdma.start()
            rdma.wait_send()

        @pl.when(my_x == 1)
        def _():
            rdma.wait_recv()

            def corr(t, carry):
                comm_ref[...] = comm_ref[...] * da_ref[...][None]
                y_ref[t] = y_ref[t] + (comm_ref[...] * c_ref[t]).sum(axis=1)
                return carry

            lax.fori_loop(0, T_CORR, corr, 0)

    yT = pl.pallas_call(
        body,
        out_shape=jax.ShapeDtypeStruct((S, Bb, D), jnp.float32),
        in_specs=[pl.BlockSpec(memory_space=pltpu.VMEM)] * 4,
        out_specs=pl.BlockSpec(memory_space=pltpu.VMEM),
        scratch_shapes=[
            pltpu.VMEM((Bb, N, D), jnp.float32),
            pltpu.VMEM((Bb, N, D), jnp.float32),
            pltpu.SemaphoreType.DMA,
            pltpu.SemaphoreType.DMA,
        ],
    )(xT, dAT, Bq, Cq)

    return yT.transpose(1, 0, 2)


# baseline (device time: 277193 ns/iter reference)
import jax
import jax.numpy as jnp
from jax import lax
from jax.experimental import pallas as pl
from jax.experimental.pallas import tpu as pltpu

T_CORR = 160


def kernel(x, A, B, C):
    Bb, S, D = x.shape
    N = A.shape[1]

    dAT = jnp.exp(A).T
    xT = x.transpose(1, 0, 2)
    Bq = B.transpose(1, 0, 2)
    Cq = C.transpose(1, 0, 2)

    def body(x_ref, da_ref, b_ref, c_ref, y_ref, h_ref, comm_ref,
             send_sem, recv_sem):
        my_x = lax.axis_index("x")
        my_y = lax.axis_index("y")

        h_ref[...] = jnp.zeros(h_ref.shape, h_ref.dtype)
        da = da_ref[...][None]

        def step(t, carry):
            h = h_ref[...] * da + x_ref[t][:, None, :] * b_ref[t][:, :, None]
            h_ref[...] = h
            y_ref[t] = (h * c_ref[t][:, :, None]).sum(axis=1)
            return carry

        lax.fori_loop(0, S, step, 0)

        rdma = pltpu.make_async_remote_copy(
            src_ref=h_ref,
            dst_ref=comm_ref,
            send_sem=send_sem,
            recv_sem=recv_sem,
            device_id=(1 - my_x, my_y),
            device_id_type=pl.DeviceIdType.MESH,
        )

        @pl.when(my_x == 0)
        def _():
            rdma.start()
            rdma.wait_send()

        @pl.when(my_x == 1)
        def _():
            rdma.wait_recv()

            def corr(t, carry):
                comm_ref[...] = comm_ref[...] * da_ref[...][None]
                y_ref[t] = y_ref[t] + (comm_ref[...] * c_ref[t][:, :, None]).sum(axis=1)
                return carry

            lax.fori_loop(0, T_CORR, corr, 0)

    yT = pl.pallas_call(
        body,
        out_shape=jax.ShapeDtypeStruct((S, Bb, D), jnp.float32),
        in_specs=[pl.BlockSpec(memory_space=pltpu.VMEM)] * 4,
        out_specs=pl.BlockSpec(memory_space=pltpu.VMEM),
        scratch_shapes=[
            pltpu.VMEM((Bb, N, D), jnp.float32),
            pltpu.VMEM((Bb, N, D), jnp.float32),
            pltpu.SemaphoreType.DMA,
            pltpu.SemaphoreType.DMA,
        ],
    )(xT, dAT, Bq, Cq)

    return yT.transpose(1, 0, 2)


# device time: 236297 ns/iter; 1.1731x vs baseline; 1.1731x over previous
import jax
import jax.numpy as jnp
from jax import lax
from jax.experimental import pallas as pl
from jax.experimental.pallas import tpu as pltpu

T_CORR = 160


def kernel(x, A, B, C):
    Bb, S, D = x.shape
    N = A.shape[1]

    dAT = jnp.exp(A).T
    xT = x.transpose(1, 0, 2)
    Bq = B.transpose(1, 0, 2)
    Cq = C.transpose(1, 0, 2)

    def body(x_ref, da_ref, b_ref, c_ref, y_ref, h_ref, comm_ref,
             send_sem, recv_sem):
        my_x = lax.axis_index("x")
        my_y = lax.axis_index("y")

        da = da_ref[...][None]

        UNROLL = 8

        def step(i, h):
            for k in range(UNROLL):
                t = i * UNROLL + k
                h = h * da + x_ref[t][:, None, :] * b_ref[t][:, :, None]
                y_ref[t] = (h * c_ref[t][:, :, None]).sum(axis=1)
            return h

        h0 = jnp.zeros(h_ref.shape, h_ref.dtype)
        h_ref[...] = lax.fori_loop(0, S // UNROLL, step, h0)

        rdma = pltpu.make_async_remote_copy(
            src_ref=h_ref,
            dst_ref=comm_ref,
            send_sem=send_sem,
            recv_sem=recv_sem,
            device_id=(1 - my_x, my_y),
            device_id_type=pl.DeviceIdType.MESH,
        )

        @pl.when(my_x == 0)
        def _():
            rdma.start()
            rdma.wait_send()

        @pl.when(my_x == 1)
        def _():
            rdma.wait_recv()

            def corr(i, hc):
                for k in range(UNROLL):
                    t = i * UNROLL + k
                    hc = hc * da[0]
                    y_ref[t] = y_ref[t] + (hc * c_ref[t][:, :, None]).sum(axis=1)
                return hc

            lax.fori_loop(0, T_CORR // UNROLL, corr, comm_ref[...])

    yT = pl.pallas_call(
        body,
        out_shape=jax.ShapeDtypeStruct((S, Bb, D), jnp.float32),
        in_specs=[pl.BlockSpec(memory_space=pltpu.VMEM)] * 4,
        out_specs=pl.BlockSpec(memory_space=pltpu.VMEM),
        scratch_shapes=[
            pltpu.VMEM((Bb, N, D), jnp.float32),
            pltpu.VMEM((Bb, N, D), jnp.float32),
            pltpu.SemaphoreType.DMA,
            pltpu.SemaphoreType.DMA,
        ],
    )(xT, dAT, Bq, Cq)

    return yT.transpose(1, 0, 2)


# device time: 167079 ns/iter; 1.6591x vs baseline; 1.4143x over previous
import jax
import jax.numpy as jnp
from jax import lax
from jax.experimental import pallas as pl
from jax.experimental.pallas import tpu as pltpu

T_CORR = 160
UNROLL = 8

_DOT_DIMS = (((1,), (1,)), ((0,), (0,)))


def kernel(x, A, B, C):
    Bb, S, D = x.shape
    N = A.shape[1]

    bf16 = jnp.bfloat16
    dAT = jnp.exp(A).T.astype(bf16)
    xT = x.transpose(1, 0, 2).astype(bf16)
    Bq = B.transpose(1, 0, 2).astype(bf16)
    Cq = C.transpose(1, 0, 2).astype(bf16)

    def body(x_ref, da_ref, b_ref, c_ref, y_ref, h_ref, comm_ref,
             send_sem, recv_sem):
        my_x = lax.axis_index("x")
        my_y = lax.axis_index("y")

        da = da_ref[...][None]

        def step(i, h):
            for k in range(UNROLL):
                t = i * UNROLL + k
                h = h * da + x_ref[t][:, None, :] * b_ref[t][:, :, None]
                y_ref[t] = lax.dot_general(
                    c_ref[t], h, _DOT_DIMS,
                    preferred_element_type=jnp.float32,
                )
            return h

        h0 = jnp.zeros(h_ref.shape, h_ref.dtype)
        h_ref[...] = lax.fori_loop(0, S // UNROLL, step, h0)

        rdma = pltpu.make_async_remote_copy(
            src_ref=h_ref,
            dst_ref=comm_ref,
            send_sem=send_sem,
            recv_sem=recv_sem,
            device_id=(1 - my_x, my_y),
            device_id_type=pl.DeviceIdType.MESH,
        )

        @pl.when(my_x == 0)
        def _():
            rdma.start()
            rdma.wait_send()

        @pl.when(my_x == 1)
        def _():
            rdma.wait_recv()

            def corr(i, hc):
                for k in range(UNROLL):
                    t = i * UNROLL + k
                    hc = hc * da[0]
                    y_ref[t] = y_ref[t] + lax.dot_general(
                        c_ref[t], hc, _DOT_DIMS,
                        preferred_element_type=jnp.float32,
                    )
                return hc

            lax.fori_loop(0, T_CORR // UNROLL, corr, comm_ref[...])

    yT = pl.pallas_call(
        body,
        out_shape=jax.ShapeDtypeStruct((S, Bb, D), jnp.float32),
        in_specs=[pl.BlockSpec(memory_space=pltpu.VMEM)] * 4,
        out_specs=pl.BlockSpec(memory_space=pltpu.VMEM),
        scratch_shapes=[
            pltpu.VMEM((Bb, N, D), bf16),
            pltpu.VMEM((Bb, N, D), bf16),
            pltpu.SemaphoreType.DMA,
            pltpu.SemaphoreType.DMA,
        ],
    )(xT, dAT, Bq, Cq)

    return yT.transpose(1, 0, 2)
